# Initial kernel scaffold; baseline (speedup 1.0000x reference)
import jax
import jax.numpy as jnp
from jax.experimental import pallas as pl


def kernel(input, W):
    raise NotImplementedError("write your pallas kernel here")


# speedup vs baseline: 1.3900x; 1.3900x over previous
import functools

import jax
import jax.numpy as jnp
from jax import lax
from jax.experimental import pallas as pl
from jax.experimental.pallas import tpu as pltpu
from jax.experimental.pallas import tpu_sc as plsc

D = 64
NC, NS = 2, 16
NW = NC * NS
SUB = 128
NSUB = 4
CHUNK = SUB * NSUB


@functools.partial(jax.jit, static_argnums=(2,))
def _gather(W, ids2d, B):
    b_per_w = B // NW
    n_chunks = b_per_w // CHUNK

    mesh = plsc.VectorSubcoreMesh(core_axis_name="c", subcore_axis_name="s")

    @functools.partial(
        pl.kernel,
        out_type=jax.ShapeDtypeStruct((B, D), jnp.float32),
        mesh=mesh,
        scratch_types=[
            pltpu.VMEM((NSUB, SUB), jnp.int32),
            pltpu.VMEM((CHUNK, D), jnp.float32),
            pltpu.SemaphoreType.DMA,
        ],
        compiler_params=pltpu.CompilerParams(use_tc_tiling_on_sc=False),
    )
    def body(table_hbm, ids_hbm, out_hbm, idx_v, rows_v, gsem):
        wid = lax.axis_index("s") * NC + lax.axis_index("c")
        idx_row0 = wid * (b_per_w // SUB)
        out_row0 = wid * b_per_w

        def chunk(g, carry):
            r0 = pl.multiple_of(idx_row0 + g * NSUB, NSUB)
            pltpu.sync_copy(ids_hbm.at[pl.ds(r0, NSUB)], idx_v)
            cps = []
            for j in range(NSUB):
                cps.append(pltpu.async_copy(
                    table_hbm.at[idx_v.at[j]],
                    rows_v.at[pl.ds(j * SUB, SUB)], gsem))
            for cp in cps:
                cp.wait()
            o0 = pl.multiple_of(out_row0 + g * CHUNK, CHUNK)
            pltpu.sync_copy(rows_v, out_hbm.at[pl.ds(o0, CHUNK)])
            return carry

        lax.fori_loop(0, n_chunks, chunk, 0)

    return body(W, ids2d)


def kernel(input, W):
    flat = input.reshape(-1).astype(jnp.int32)
    B = flat.shape[0]
    ids2d = flat.reshape(B // SUB, SUB)
    return _gather(W, ids2d, B)


# speedup vs baseline: 1.4515x; 1.0442x over previous
import functools

import jax
import jax.numpy as jnp
from jax import lax
from jax.experimental import pallas as pl
from jax.experimental.pallas import tpu as pltpu
from jax.experimental.pallas import tpu_sc as plsc

D = 64
NC, NS = 2, 16
NW = NC * NS
SUB = 128
NSUB = 2
CHUNK = SUB * NSUB
NBUF = 4


@functools.partial(jax.jit, static_argnums=(2,))
def _gather(W, ids2d, B):
    b_per_w = B // NW
    n_chunks = b_per_w // CHUNK
    assert n_chunks % NBUF == 0 and n_chunks >= 2 * NBUF

    mesh = plsc.VectorSubcoreMesh(core_axis_name="c", subcore_axis_name="s")

    @functools.partial(
        pl.kernel,
        out_type=jax.ShapeDtypeStruct((B, D), jnp.float32),
        mesh=mesh,
        scratch_types=[
            pltpu.VMEM((NBUF, NSUB, SUB), jnp.int32),
            pltpu.VMEM((NBUF, CHUNK, D), jnp.float32),
            pltpu.SemaphoreType.DMA((NBUF,)),
            pltpu.SemaphoreType.DMA((NBUF,)),
            pltpu.SemaphoreType.DMA((NBUF,)),
        ],
        compiler_params=pltpu.CompilerParams(use_tc_tiling_on_sc=False),
    )
    def body(table_hbm, ids_hbm, out_hbm, idx_v, rows_v, isem, gsem, osem):
        wid = lax.axis_index("s") * NC + lax.axis_index("c")
        idx_row0 = wid * (b_per_w // SUB)
        out_row0 = wid * b_per_w

        def idx_cp(b, g):
            r0 = idx_row0 + g * NSUB
            return pltpu.make_async_copy(
                ids_hbm.at[pl.ds(r0, NSUB)], idx_v.at[b], isem.at[b])

        def gath_cp(b, j):
            return pltpu.make_async_copy(
                table_hbm.at[idx_v.at[b, j]],
                rows_v.at[b, pl.ds(j * SUB, SUB)], gsem.at[b])

        def store_cp(b, g):
            o0 = out_row0 + g * CHUNK
            return pltpu.make_async_copy(
                rows_v.at[b], out_hbm.at[pl.ds(o0, CHUNK)], osem.at[b])

        def stage_b(b, g):
            store_cp(b, g - NBUF).wait()
            idx_cp(b, g).wait()
            for j in range(NSUB):
                gath_cp(b, j).start()

        def stage_c(b, g):
            for j in range(NSUB):
                gath_cp(b, j).wait()
            store_cp(b, g).start()
            idx_cp(b, (g + NBUF) % n_chunks).start()

        for b in range(NBUF):
            idx_cp(b, b).start()
        for b in range(NBUF):
            idx_cp(b, b).wait()
            for j in range(NSUB):
                gath_cp(b, j).start()
        for b in range(2):
            stage_c(b, b)

        def step(s, carry):
            for k in range(NBUF):
                g = NBUF * s + 2 + k
                stage_b(k, g + 2)
                stage_c((2 + k) % NBUF, g)
            return carry

        lax.fori_loop(0, (n_chunks - NBUF) // NBUF, step, 0)

        for k in range(2):
            g = n_chunks - 2 + k
            stage_c(g % NBUF, g)
        for b in range(NBUF):
            g = n_chunks - NBUF + b
            store_cp(b, g).wait()
            idx_cp(b, b).wait()

    return body(W, ids2d)


def kernel(input, W):
    flat = input.reshape(-1).astype(jnp.int32)
    B = flat.shape[0]
    ids2d = flat.reshape(B // SUB, SUB)
    return _gather(W, ids2d, B)


# speedup vs baseline: 1.9303x; 1.3299x over previous
import functools

import jax
import jax.numpy as jnp
from jax import lax
from jax.experimental import pallas as pl
from jax.experimental.pallas import tpu as pltpu
from jax.experimental.pallas import tpu_sc as plsc

D = 64
NC, NS = 2, 16
NW = NC * NS
SUB = 128
NSUB = 2
CHUNK = SUB * NSUB
NBUF = 4


@functools.partial(jax.jit, static_argnums=(2,))
def _gather(W, ids2d, B):
    b_per_w = B // NW
    n_chunks = b_per_w // CHUNK
    assert n_chunks % NBUF == 0 and n_chunks >= 2 * NBUF

    mesh = plsc.VectorSubcoreMesh(core_axis_name="c", subcore_axis_name="s")

    @functools.partial(
        pl.kernel,
        out_type=jax.ShapeDtypeStruct((B, 2 * D), jnp.float32),
        mesh=mesh,
        scratch_types=[
            pltpu.VMEM((NBUF, NSUB, SUB), jnp.int32),
            pltpu.VMEM((NBUF, CHUNK, D), jnp.float32),
            pltpu.SemaphoreType.DMA((NBUF,)),
            pltpu.SemaphoreType.DMA((NBUF,)),
            pltpu.SemaphoreType.DMA((NBUF,)),
        ],
        compiler_params=pltpu.CompilerParams(use_tc_tiling_on_sc=False),
    )
    def body(table_hbm, ids_hbm, out_hbm, idx_v, rows_v, isem, gsem, osem):
        wid = lax.axis_index("s") * NC + lax.axis_index("c")
        idx_row0 = wid * (b_per_w // SUB)
        out_row0 = wid * b_per_w

        def idx_cp(b, g):
            r0 = idx_row0 + g * NSUB
            return pltpu.make_async_copy(
                ids_hbm.at[pl.ds(r0, NSUB)], idx_v.at[b], isem.at[b])

        def gath_cp(b, j):
            return pltpu.make_async_copy(
                table_hbm.at[idx_v.at[b, j]],
                rows_v.at[b, pl.ds(j * SUB, SUB)], gsem.at[b])

        def store_cp(b, g):
            o0 = out_row0 + g * CHUNK
            return pltpu.make_async_copy(
                rows_v.at[b],
                out_hbm.at[pl.ds(o0, CHUNK), pl.ds(0, D)], osem.at[b])

        def stage_b(b, g):
            store_cp(b, g - NBUF).wait()
            idx_cp(b, g).wait()
            for j in range(NSUB):
                gath_cp(b, j).start()

        def stage_c(b, g):
            for j in range(NSUB):
                gath_cp(b, j).wait()
            store_cp(b, g).start()
            idx_cp(b, (g + NBUF) % n_chunks).start()

        for b in range(NBUF):
            idx_cp(b, b).start()
        for b in range(NBUF):
            idx_cp(b, b).wait()
            for j in range(NSUB):
                gath_cp(b, j).start()
        for b in range(2):
            stage_c(b, b)

        def step(s, carry):
            for k in range(NBUF):
                g = NBUF * s + 2 + k
                stage_b(k, g + 2)
                stage_c((2 + k) % NBUF, g)
            return carry

        lax.fori_loop(0, (n_chunks - NBUF) // NBUF, step, 0)

        for k in range(2):
            g = n_chunks - 2 + k
            stage_c(g % NBUF, g)
        for b in range(NBUF):
            g = n_chunks - NBUF + b
            store_cp(b, g).wait()
            idx_cp(b, b).wait()

    return body(W, ids2d)[:, :D]


def kernel(input, W):
    flat = input.reshape(-1).astype(jnp.int32)
    B = flat.shape[0]
    ids2d = flat.reshape(B // SUB, SUB)
    return _gather(W, ids2d, B)


# speedup vs baseline: 2.0733x; 1.0740x over previous
import functools

import jax
import jax.numpy as jnp
from jax import lax
from jax.experimental import pallas as pl
from jax.experimental.pallas import tpu as pltpu
from jax.experimental.pallas import tpu_sc as plsc

D = 64
NC, NS = 2, 16
NW = NC * NS
SUB = 128
NSUB = 2
CHUNK = SUB * NSUB
NBUF = 4


@functools.partial(jax.jit, static_argnums=(2,))
def _gather(W, ids2d, B):
    b_per_w = B // NW
    n_chunks = b_per_w // CHUNK
    assert n_chunks % NBUF == 0 and n_chunks >= 2 * NBUF

    mesh = plsc.VectorSubcoreMesh(core_axis_name="c", subcore_axis_name="s")

    @functools.partial(
        pl.kernel,
        out_type=jax.ShapeDtypeStruct((B, 2 * D), jnp.float32),
        mesh=mesh,
        scratch_types=[
            pltpu.VMEM((NBUF, NSUB, SUB), jnp.int32),
            pltpu.VMEM((NBUF, CHUNK, D), jnp.float32),
            pltpu.SemaphoreType.DMA((NBUF,)),
            pltpu.SemaphoreType.DMA((NBUF,)),
            pltpu.SemaphoreType.DMA((NBUF,)),
        ],
        compiler_params=pltpu.CompilerParams(use_tc_tiling_on_sc=False),
    )
    def body(table_hbm, ids_hbm, out_hbm, idx_v, rows_v, isem, gsem, osem):
        wid = lax.axis_index("s") * NC + lax.axis_index("c")
        idx_row0 = wid * (b_per_w // SUB)
        out_row0 = wid * b_per_w

        def idx_cp(b, g):
            r0 = idx_row0 + g * NSUB
            return pltpu.make_async_copy(
                ids_hbm.at[pl.ds(r0, NSUB)], idx_v.at[b], isem.at[b])

        def gath_cp(b, j):
            return pltpu.make_async_copy(
                table_hbm.at[idx_v.at[b, j]],
                rows_v.at[b, pl.ds(j * SUB, SUB)], gsem.at[b])

        def store_cp(b, g):
            o0 = out_row0 + g * CHUNK
            return pltpu.make_async_copy(
                rows_v.at[b],
                out_hbm.at[pl.ds(o0, CHUNK), pl.ds(0, D)], osem.at[b])

        def stage_b(b, g):
            store_cp(b, g - NBUF).wait()
            idx_cp(b, g).wait()
            for j in range(NSUB):
                gath_cp(b, j).start()

        def stage_c(b, g):
            for j in range(NSUB):
                gath_cp(b, j).wait()
            store_cp(b, g).start()
            idx_cp(b, (g + NBUF) % n_chunks).start()

        for b in range(NBUF):
            idx_cp(b, b).start()
        for b in range(NBUF):
            idx_cp(b, b).wait()
            for j in range(NSUB):
                gath_cp(b, j).start()
        for b in range(2):
            stage_c(b, b)

        def step(s, carry):
            for k in range(NBUF):
                g = NBUF * s + 2 + k
                stage_b(k, g + 2)
                stage_c((2 + k) % NBUF, g)
            return carry

        lax.fori_loop(0, (n_chunks - NBUF) // NBUF, step, 0)

        for k in range(2):
            g = n_chunks - 2 + k
            stage_c(g % NBUF, g)
        for b in range(NBUF):
            g = n_chunks - NBUF + b
            store_cp(b, g).wait()
            idx_cp(b, b).wait()

    return body(W, ids2d)[:, :D]


def kernel(input, W):
    flat = input.reshape(-1).astype(jnp.int32) * 2
    B = flat.shape[0]
    ids2d = flat.reshape(B // SUB, SUB)
    W2 = jnp.pad(W, ((0, 0), (0, D))).reshape(2 * W.shape[0], D)
    return _gather(W2, ids2d, B)


# speedup vs baseline: 2.0749x; 1.0008x over previous
import functools

import jax
import jax.numpy as jnp
from jax import lax
from jax.experimental import pallas as pl
from jax.experimental.pallas import tpu as pltpu
from jax.experimental.pallas import tpu_sc as plsc

D = 64
NC, NS = 2, 16
NW = NC * NS
SUB = 128
NSUB = 2
CHUNK = SUB * NSUB
NBUF = 4


@functools.partial(jax.jit, static_argnums=(2,))
def _gather(W, ids2d, B):
    b_per_w = B // NW
    n_chunks = b_per_w // CHUNK
    assert n_chunks % NBUF == 0 and n_chunks >= 2 * NBUF

    mesh = plsc.VectorSubcoreMesh(core_axis_name="c", subcore_axis_name="s")

    @functools.partial(
        pl.kernel,
        out_type=jax.ShapeDtypeStruct((B, 2 * D), jnp.float32),
        mesh=mesh,
        scratch_types=[
            pltpu.VMEM((NBUF, NSUB, SUB), jnp.int32),
            pltpu.VMEM((NBUF, CHUNK, D), jnp.float32),
            pltpu.SemaphoreType.DMA((NBUF,)),
            pltpu.SemaphoreType.DMA((NBUF,)),
            pltpu.SemaphoreType.DMA((NBUF,)),
        ],
        compiler_params=pltpu.CompilerParams(use_tc_tiling_on_sc=False),
    )
    def body(table_hbm, ids_hbm, out_hbm, idx_v, rows_v, isem, gsem, osem):
        wid = lax.axis_index("s") * NC + lax.axis_index("c")
        idx_row0 = wid * (b_per_w // SUB)
        out_row0 = wid * b_per_w

        def idx_cp(b, g):
            r0 = idx_row0 + g * NSUB
            return pltpu.make_async_copy(
                ids_hbm.at[pl.ds(r0, NSUB)], idx_v.at[b], isem.at[b])

        def gath_cp(b, j):
            return pltpu.make_async_copy(
                table_hbm.at[idx_v.at[b, j]],
                rows_v.at[b, pl.ds(j * SUB, SUB)], gsem.at[b])

        def store_cp(b, g):
            o0 = out_row0 + g * CHUNK
            return pltpu.make_async_copy(
                rows_v.at[b],
                out_hbm.at[pl.ds(o0, CHUNK), pl.ds(0, D)], osem.at[b])

        def stage_b(b, g):
            store_cp(b, g - NBUF).wait()
            idx_cp(b, g).wait()
            for j in range(NSUB):
                gath_cp(b, j).start()

        def stage_c(b, g):
            for j in range(NSUB):
                gath_cp(b, j).wait()
            store_cp(b, g).start()
            idx_cp(b, (g + NBUF) % n_chunks).start()

        for b in range(NBUF):
            idx_cp(b, b).start()
        for b in range(NBUF):
            idx_cp(b, b).wait()
            for j in range(NSUB):
                gath_cp(b, j).start()
        for b in range(2):
            stage_c(b, b)

        def step(s, carry):
            for k in range(NBUF):
                g = NBUF * s + 2 + k
                stage_b(k, g + 2)
                stage_c((2 + k) % NBUF, g)
            return carry

        lax.fori_loop(0, (n_chunks - NBUF) // NBUF, step, 0)

        for k in range(2):
            g = n_chunks - 2 + k
            stage_c(g % NBUF, g)
        for b in range(NBUF):
            g = n_chunks - NBUF + b
            store_cp(b, g).wait()
            idx_cp(b, b).wait()

    return body(W, ids2d)[:, :D]


def kernel(input, W):
    flat = input.reshape(-1).astype(jnp.int32) * 2
    B = flat.shape[0]
    ids2d = flat.reshape(B // SUB, SUB)
    V = W.shape[0]
    W2 = jnp.pad(W.reshape(V // 8, 8, D),
                 ((0, 0), (0, 0), (0, D))).reshape(2 * V, D)
    return _gather(W2, ids2d, B)
